# Initial kernel scaffold; baseline (speedup 1.0000x reference)
import jax
import jax.numpy as jnp
from jax.experimental import pallas as pl


def kernel(rois, fg_scores, gts):
    raise NotImplementedError("write your pallas kernel here")


# speedup vs baseline: 14.8520x; 14.8520x over previous
import jax
import jax.numpy as jnp
from jax.experimental import pallas as pl
from jax.experimental.pallas import tpu as pltpu

_POS_T = 0.7
_NEG_T = 0.3
_NFG = 128
_NBG = 128
_N = 5000
_R = 40
_C = 128
_G = 64


def _smooth_l1(v):
    av = jnp.abs(v)
    return jnp.where(av < 1.0, 0.5 * av * av, av - 0.5)


def _tc_body(coords_ref, scores_ref, gts_ref, out_ref, iou_ref):
    x1 = coords_ref[0]
    y1 = coords_ref[1]
    x2 = coords_ref[2]
    y2 = coords_ref[3]
    w = x2 - x1 + 1.0
    h = y2 - y1 + 1.0
    area = w * h
    rcx = x1 + 0.5 * w
    rcy = y1 + 0.5 * h

    max_ov = jnp.full((_R, _C), -1.0, jnp.float32)
    argm = jnp.zeros((_R, _C), jnp.int32)
    cms = []
    for j in range(_G):
        gx1 = gts_ref[0, j]
        gy1 = gts_ref[1, j]
        gx2 = gts_ref[2, j]
        gy2 = gts_ref[3, j]
        ga = (gx2 - gx1 + 1.0) * (gy2 - gy1 + 1.0)
        iw = jnp.maximum(jnp.minimum(x2, gx2) - jnp.maximum(x1, gx1) + 1.0, 0.0)
        ih = jnp.maximum(jnp.minimum(y2, gy2) - jnp.maximum(y1, gy1) + 1.0, 0.0)
        inter = iw * ih
        iou = inter / (area + ga - inter)
        iou_ref[j] = iou
        upd = iou > max_ov
        argm = jnp.where(upd, j, argm)
        max_ov = jnp.where(upd, iou, max_ov)
        cms.append(jnp.max(iou))

    keep = jnp.zeros((_R, _C), jnp.bool_)
    tcx = jnp.zeros((_R, _C), jnp.float32)
    tcy = jnp.zeros((_R, _C), jnp.float32)
    tw = jnp.zeros((_R, _C), jnp.float32)
    th = jnp.zeros((_R, _C), jnp.float32)
    for j in range(_G):
        cm = cms[j]
        cm = jnp.where(cm == 0.0, 1e-5, cm)
        keep = keep | (iou_ref[j] == cm)
        gx1 = gts_ref[0, j]
        gy1 = gts_ref[1, j]
        gx2 = gts_ref[2, j]
        gy2 = gts_ref[3, j]
        gw = gx2 - gx1 + 1.0
        gh = gy2 - gy1 + 1.0
        gcx = gx1 + 0.5 * gw
        gcy = gy1 + 0.5 * gh
        m = argm == j
        tcx = jnp.where(m, gcx, tcx)
        tcy = jnp.where(m, gcy, tcy)
        tw = jnp.where(m, gw, tw)
        th = jnp.where(m, gh, th)

    fi = (jax.lax.broadcasted_iota(jnp.int32, (_R, _C), 0) * _C
          + jax.lax.broadcasted_iota(jnp.int32, (_R, _C), 1))
    valid = fi < _N
    pos = (keep | (max_ov > _POS_T)) & valid
    neg = (max_ov < _NEG_T) & (~pos) & valid
    posf = pos.astype(jnp.float32)
    negf = neg.astype(jnp.float32)

    ik = jax.lax.broadcasted_iota(jnp.int32, (_C, _C), 0)
    ic = jax.lax.broadcasted_iota(jnp.int32, (_C, _C), 1)
    tri_c = (ik < ic).astype(jnp.float32)
    ik40 = jax.lax.broadcasted_iota(jnp.int32, (_R, _R), 0)
    ic40 = jax.lax.broadcasted_iota(jnp.int32, (_R, _R), 1)
    tri_r = (ic40 < ik40).astype(jnp.float32)

    inrow_p = jnp.dot(posf, tri_c, preferred_element_type=jnp.float32)
    inrow_n = jnp.dot(negf, tri_c, preferred_element_type=jnp.float32)
    rs_p = jnp.broadcast_to(jnp.sum(posf, axis=1, keepdims=True), (_R, _C))
    rs_n = jnp.broadcast_to(jnp.sum(negf, axis=1, keepdims=True), (_R, _C))
    rowpref_p = jnp.dot(tri_r, rs_p, preferred_element_type=jnp.float32)
    rowpref_n = jnp.dot(tri_r, rs_n, preferred_element_type=jnp.float32)
    excl_p = rowpref_p + inrow_p
    excl_n = rowpref_n + inrow_n
    selp = posf * (excl_p < float(_NFG)).astype(jnp.float32)
    seln = negf * (excl_n < float(_NBG)).astype(jnp.float32)

    loc_i = (_smooth_l1(rcx - tcx) + _smooth_l1(rcy - tcy)
             + _smooth_l1(w - tw) + _smooth_l1(h - th))
    s = scores_ref[...]
    sp = jnp.log(1.0 + jnp.exp(-jnp.abs(s)))
    relu = jnp.maximum(s, 0.0)
    bce1 = relu - s + sp
    bce0 = relu + sp

    e00 = (fi == 0).astype(jnp.float32)
    m0p = jnp.sum(posf * e00)
    m0n = jnp.sum(negf * e00)
    loc00 = jnp.sum(loc_i * e00)
    b1_00 = jnp.sum(bce1 * e00)
    b0_00 = jnp.sum(bce0 * e00)

    p_tot = jnp.sum(posf)
    n_tot = jnp.sum(negf)
    kp = jnp.minimum(p_tot, float(_NFG))
    kn = jnp.minimum(n_tot, float(_NBG))
    padp = float(_NFG) - kp
    padn = float(_NBG) - kn
    sum_pos_w = kp + padp * m0p
    sum_neg_w = kn + padn * m0n

    loc_num = jnp.sum(selp * loc_i) + padp * m0p * loc00
    loc_loss = loc_num / jnp.maximum(sum_pos_w * 4.0, 1.0)
    cls_p = (jnp.sum(selp * bce1) + padp * m0p * b1_00) / jnp.maximum(sum_pos_w, 1.0)
    cls_n = (jnp.sum(seln * bce0) + padn * m0n * b0_00) / jnp.maximum(sum_neg_w, 1.0)
    out_ref[0, 0] = loc_loss + cls_p + cls_n


@jax.jit
def kernel(rois, fg_scores, gts):
    rois_pad = jnp.full((_R * _C, 4), -1e5, jnp.float32).at[:_N].set(rois)
    coords = rois_pad.T.reshape(4, _R, _C)
    scores = jnp.pad(fg_scores[:, 0], (0, _R * _C - _N)).reshape(_R, _C)
    gts_t = gts.T[:4]
    out = pl.pallas_call(
        _tc_body,
        out_shape=jax.ShapeDtypeStruct((1, 1), jnp.float32),
        in_specs=[
            pl.BlockSpec(memory_space=pltpu.MemorySpace.VMEM),
            pl.BlockSpec(memory_space=pltpu.MemorySpace.VMEM),
            pl.BlockSpec(memory_space=pltpu.MemorySpace.SMEM),
        ],
        out_specs=pl.BlockSpec(memory_space=pltpu.MemorySpace.SMEM),
        scratch_shapes=[pltpu.VMEM((_G, _R, _C), jnp.float32)],
    )(coords, scores, gts_t)
    return out[0, 0]
